# Initial kernel scaffold; baseline (speedup 1.0000x reference)
import functools

import jax
import jax.numpy as jnp
from jax import lax
from jax.experimental import pallas as pl
from jax.experimental.pallas import tpu as pltpu
from jax.experimental.pallas import tpu_sc as plsc

N = 10000
E = 320000
D = 128

NC = 2
NS = 16
NW = NC * NS

E_TILE = E // NW
CHUNK = 128
N_FULL = E_TILE // CHUNK
REM = E_TILE - N_FULL * CHUNK

ROWS_SUB = N // NS

_mesh = plsc.VectorSubcoreMesh(core_axis_name="c", subcore_axis_name="s")


@functools.partial(
    pl.kernel,
    out_type=jax.ShapeDtypeStruct((NW, N), jnp.float32),
    mesh=_mesh,
    scratch_types=[
        pltpu.VMEM((CHUNK,), jnp.int32),
        pltpu.VMEM((REM,), jnp.int32),
        pltpu.VMEM((N,), jnp.float32),
    ],
)
def _deg_sc(row_hbm, out_hbm, idx_v, ridx_v, hist_v):
    c = lax.axis_index("c")
    s = lax.axis_index("s")
    wid = s * NC + c
    base = wid * E_TILE

    zeros16 = jnp.zeros((16,), jnp.float32)

    def zbody(i, carry):
        hist_v[pl.ds(i * 16, 16)] = zeros16
        return carry

    lax.fori_loop(0, N // 16, zbody, 0)

    ones16 = jnp.ones((16,), jnp.float32)

    def cbody(j, carry):
        pltpu.sync_copy(row_hbm.at[pl.ds(base + j * CHUNK, CHUNK)], idx_v)
        for k in range(CHUNK // 16):
            idx16 = idx_v[pl.ds(k * 16, 16)]
            plsc.addupdate_scatter(hist_v, [idx16], ones16)
        return carry

    lax.fori_loop(0, N_FULL, cbody, 0)

    pltpu.sync_copy(row_hbm.at[pl.ds(base + N_FULL * CHUNK, REM)], ridx_v)
    plsc.addupdate_scatter(hist_v, [ridx_v[...]], ones16)

    pltpu.sync_copy(hist_v, out_hbm.at[wid])


@functools.partial(
    pl.kernel,
    out_type=jax.ShapeDtypeStruct((NC, N, D), jnp.float32),
    mesh=_mesh,
    scratch_types=[
        pltpu.VMEM_SHARED((N, D), jnp.float32),
        pltpu.VMEM((CHUNK,), jnp.int32),
        pltpu.VMEM((CHUNK,), jnp.int32),
        pltpu.VMEM((CHUNK, D), jnp.float32),
        pltpu.VMEM((REM,), jnp.int32),
        pltpu.VMEM((REM,), jnp.int32),
        pltpu.VMEM((REM, D), jnp.float32),
        pltpu.SemaphoreType.DMA,
    ],
)
def _scatter_sc(hp_hbm, row_hbm, col_hbm, zeros_hbm, out_hbm,
                acc_sh, row_v, col_v, msg_v, rrow_v, rcol_v, rmsg_v, sem):
    c = lax.axis_index("c")
    s = lax.axis_index("s")
    wid = s * NC + c
    base = wid * E_TILE
    r0 = s * ROWS_SUB

    pltpu.sync_copy(zeros_hbm.at[pl.ds(r0, ROWS_SUB)], acc_sh.at[pl.ds(r0, ROWS_SUB)])
    plsc.subcore_barrier()

    def cbody(j, carry):
        eb = base + j * CHUNK
        pltpu.sync_copy(row_hbm.at[pl.ds(eb, CHUNK)], row_v)
        pltpu.sync_copy(col_hbm.at[pl.ds(eb, CHUNK)], col_v)
        pltpu.async_copy(hp_hbm.at[row_v], msg_v, sem).wait()
        pltpu.sync_copy(msg_v, acc_sh.at[col_v], add=True)
        return carry

    lax.fori_loop(0, N_FULL, cbody, 0)

    eb = base + N_FULL * CHUNK
    pltpu.sync_copy(row_hbm.at[pl.ds(eb, REM)], rrow_v)
    pltpu.sync_copy(col_hbm.at[pl.ds(eb, REM)], rcol_v)
    pltpu.async_copy(hp_hbm.at[rrow_v], rmsg_v, sem).wait()
    pltpu.sync_copy(rmsg_v, acc_sh.at[rcol_v], add=True)

    plsc.subcore_barrier()
    pltpu.sync_copy(acc_sh.at[pl.ds(r0, ROWS_SUB)], out_hbm.at[c, pl.ds(r0, ROWS_SUB)])


BN = 1000
GRID = N // BN


def _dis_from_hist(hist_blk):
    deg = jnp.sum(hist_blk, axis=0) + 1.0
    return lax.rsqrt(deg)


def _pre_body(hist_ref, x_ref, w1_ref, b1_ref, out_ref):
    dis = _dis_from_hist(hist_ref[...])
    h = lax.dot_general(x_ref[...], w1_ref[...], (((1,), (1,)), ((), ())),
                        preferred_element_type=jnp.float32) + b1_ref[...]
    out_ref[...] = dis[:, None] * h


_pre_tc = pl.pallas_call(
    _pre_body,
    grid=(GRID,),
    in_specs=[
        pl.BlockSpec((NW, BN), lambda i: (0, i)),
        pl.BlockSpec((BN, D), lambda i: (i, 0)),
        pl.BlockSpec((D, D), lambda i: (0, 0)),
        pl.BlockSpec((1, D), lambda i: (0, 0)),
    ],
    out_specs=pl.BlockSpec((BN, D), lambda i: (i, 0)),
    out_shape=jax.ShapeDtypeStruct((N, D), jnp.float32),
)


def _mid_body(hist_ref, p_ref, hp_ref, w2_ref, b2_ref, out_ref):
    dis = _dis_from_hist(hist_ref[...])[:, None]
    sacc = p_ref[0] + p_ref[1] + hp_ref[...]
    y = jnp.maximum(dis * sacc, 0.0)
    h2 = lax.dot_general(y, w2_ref[...], (((1,), (1,)), ((), ())),
                         preferred_element_type=jnp.float32) + b2_ref[...]
    out_ref[...] = dis * h2


_mid_tc = pl.pallas_call(
    _mid_body,
    grid=(GRID,),
    in_specs=[
        pl.BlockSpec((NW, BN), lambda i: (0, i)),
        pl.BlockSpec((NC, BN, D), lambda i: (0, i, 0)),
        pl.BlockSpec((BN, D), lambda i: (i, 0)),
        pl.BlockSpec((D, D), lambda i: (0, 0)),
        pl.BlockSpec((1, D), lambda i: (0, 0)),
    ],
    out_specs=pl.BlockSpec((BN, D), lambda i: (i, 0)),
    out_shape=jax.ShapeDtypeStruct((N, D), jnp.float32),
)


def _post_body(hist_ref, p_ref, hp_ref, out_ref):
    dis = _dis_from_hist(hist_ref[...])[:, None]
    o = dis * (p_ref[0] + p_ref[1] + hp_ref[...])
    m = jnp.max(o, axis=1, keepdims=True)
    e = o - m
    out_ref[...] = e - jnp.log(jnp.sum(jnp.exp(e), axis=1, keepdims=True))


_post_tc = pl.pallas_call(
    _post_body,
    grid=(GRID,),
    in_specs=[
        pl.BlockSpec((NW, BN), lambda i: (0, i)),
        pl.BlockSpec((NC, BN, D), lambda i: (0, i, 0)),
        pl.BlockSpec((BN, D), lambda i: (i, 0)),
    ],
    out_specs=pl.BlockSpec((BN, D), lambda i: (i, 0)),
    out_shape=jax.ShapeDtypeStruct((N, D), jnp.float32),
)


def kernel(x, edge_index, W1, b1, W2, b2):
    row = edge_index[0]
    col = edge_index[1]
    zeros = jnp.zeros((N, D), jnp.float32)
    hist = _deg_sc(row)
    h1p = _pre_tc(hist, x, W1, b1.reshape(1, D))
    p1 = _scatter_sc(h1p, row, col, zeros)
    h2p = _mid_tc(hist, p1, h1p, W2, b2.reshape(1, D))
    p2 = _scatter_sc(h2p, row, col, zeros)
    return _post_tc(hist, p2, h2p)


# speedup vs baseline: 16.6374x; 16.6374x over previous
import functools

import jax
import jax.numpy as jnp
from jax import lax
from jax.experimental import pallas as pl
from jax.experimental.pallas import tpu as pltpu
from jax.experimental.pallas import tpu_sc as plsc

N = 10000
NP = 10240
E = 320000
D = 128

NC = 2
NS = 16
NW = NC * NS

E_TILE = E // NW
CHUNK = 128
N_FULL = E_TILE // CHUNK
REM = E_TILE - N_FULL * CHUNK

ROWS_SUB = NP // NS

_mesh = plsc.VectorSubcoreMesh(core_axis_name="c", subcore_axis_name="s")


@functools.partial(
    pl.kernel,
    out_type=jax.ShapeDtypeStruct((NW, NP), jnp.float32),
    mesh=_mesh,
    compiler_params=pltpu.CompilerParams(needs_layout_passes=False),
    scratch_types=[
        pltpu.VMEM((CHUNK,), jnp.int32),
        pltpu.VMEM((REM,), jnp.int32),
        pltpu.VMEM((NP,), jnp.float32),
    ],
)
def _deg_sc(row_hbm, out_hbm, idx_v, ridx_v, hist_v):
    c = lax.axis_index("c")
    s = lax.axis_index("s")
    wid = s * NC + c
    base = wid * E_TILE

    zeros16 = jnp.zeros((16,), jnp.float32)

    def zbody(i, carry):
        hist_v[pl.ds(i * 16, 16)] = zeros16
        return carry

    lax.fori_loop(0, NP // 16, zbody, 0)

    ones16 = jnp.ones((16,), jnp.float32)

    def cbody(j, carry):
        pltpu.sync_copy(row_hbm.at[pl.ds(base + j * CHUNK, CHUNK)], idx_v)
        for k in range(CHUNK // 16):
            idx16 = idx_v[pl.ds(k * 16, 16)]
            plsc.addupdate_scatter(hist_v, [idx16], ones16)
        return carry

    lax.fori_loop(0, N_FULL, cbody, 0)

    pltpu.sync_copy(row_hbm.at[pl.ds(base + N_FULL * CHUNK, REM)], ridx_v)
    plsc.addupdate_scatter(hist_v, [ridx_v[...]], ones16)

    pltpu.sync_copy(hist_v, out_hbm.at[wid])


@functools.partial(
    pl.kernel,
    out_type=jax.ShapeDtypeStruct((NC, NP, D), jnp.float32),
    mesh=_mesh,
    scratch_types=[
        pltpu.VMEM_SHARED((NP, D), jnp.float32),
        pltpu.VMEM((CHUNK,), jnp.int32),
        pltpu.VMEM((CHUNK,), jnp.int32),
        pltpu.VMEM((CHUNK, D), jnp.float32),
        pltpu.VMEM((REM,), jnp.int32),
        pltpu.VMEM((REM,), jnp.int32),
        pltpu.VMEM((REM, D), jnp.float32),
        pltpu.SemaphoreType.DMA,
    ],
)
def _scatter_sc(hp_hbm, row_hbm, col_hbm, zeros_hbm, out_hbm,
                acc_sh, row_v, col_v, msg_v, rrow_v, rcol_v, rmsg_v, sem):
    c = lax.axis_index("c")
    s = lax.axis_index("s")
    wid = s * NC + c
    base = wid * E_TILE
    r0 = s * ROWS_SUB

    pltpu.sync_copy(zeros_hbm.at[pl.ds(r0, ROWS_SUB)], acc_sh.at[pl.ds(r0, ROWS_SUB)])
    plsc.subcore_barrier()

    def cbody(j, carry):
        eb = base + j * CHUNK
        pltpu.sync_copy(row_hbm.at[pl.ds(eb, CHUNK)], row_v)
        pltpu.sync_copy(col_hbm.at[pl.ds(eb, CHUNK)], col_v)
        pltpu.async_copy(hp_hbm.at[row_v], msg_v, sem).wait()
        pltpu.sync_copy(msg_v, acc_sh.at[col_v], add=True)
        return carry

    lax.fori_loop(0, N_FULL, cbody, 0)

    eb = base + N_FULL * CHUNK
    pltpu.sync_copy(row_hbm.at[pl.ds(eb, REM)], rrow_v)
    pltpu.sync_copy(col_hbm.at[pl.ds(eb, REM)], rcol_v)
    pltpu.async_copy(hp_hbm.at[rrow_v], rmsg_v, sem).wait()
    pltpu.sync_copy(rmsg_v, acc_sh.at[rcol_v], add=True)

    plsc.subcore_barrier()
    pltpu.sync_copy(acc_sh.at[pl.ds(r0, ROWS_SUB)], out_hbm.at[c, pl.ds(r0, ROWS_SUB)])


BN = 1024
GRID = NP // BN


def _dis_from_hist(hist_blk):
    deg = jnp.sum(hist_blk, axis=0) + 1.0
    return lax.rsqrt(deg)


def _pre_body(hist_ref, x_ref, w1_ref, b1_ref, out_ref):
    dis = _dis_from_hist(hist_ref[...])
    h = lax.dot_general(x_ref[...], w1_ref[...], (((1,), (1,)), ((), ())),
                        preferred_element_type=jnp.float32) + b1_ref[...]
    out_ref[...] = dis[:, None] * h


_pre_tc = pl.pallas_call(
    _pre_body,
    grid=(GRID,),
    in_specs=[
        pl.BlockSpec((NW, BN), lambda i: (0, i)),
        pl.BlockSpec((BN, D), lambda i: (i, 0)),
        pl.BlockSpec((D, D), lambda i: (0, 0)),
        pl.BlockSpec((1, D), lambda i: (0, 0)),
    ],
    out_specs=pl.BlockSpec((BN, D), lambda i: (i, 0)),
    out_shape=jax.ShapeDtypeStruct((NP, D), jnp.float32),
)


def _mid_body(hist_ref, p_ref, hp_ref, w2_ref, b2_ref, out_ref):
    dis = _dis_from_hist(hist_ref[...])[:, None]
    sacc = p_ref[0] + p_ref[1] + hp_ref[...]
    y = jnp.maximum(dis * sacc, 0.0)
    h2 = lax.dot_general(y, w2_ref[...], (((1,), (1,)), ((), ())),
                         preferred_element_type=jnp.float32) + b2_ref[...]
    out_ref[...] = dis * h2


_mid_tc = pl.pallas_call(
    _mid_body,
    grid=(GRID,),
    in_specs=[
        pl.BlockSpec((NW, BN), lambda i: (0, i)),
        pl.BlockSpec((NC, BN, D), lambda i: (0, i, 0)),
        pl.BlockSpec((BN, D), lambda i: (i, 0)),
        pl.BlockSpec((D, D), lambda i: (0, 0)),
        pl.BlockSpec((1, D), lambda i: (0, 0)),
    ],
    out_specs=pl.BlockSpec((BN, D), lambda i: (i, 0)),
    out_shape=jax.ShapeDtypeStruct((NP, D), jnp.float32),
)


def _post_body(hist_ref, p_ref, hp_ref, out_ref):
    dis = _dis_from_hist(hist_ref[...])[:, None]
    o = dis * (p_ref[0] + p_ref[1] + hp_ref[...])
    m = jnp.max(o, axis=1, keepdims=True)
    e = o - m
    out_ref[...] = e - jnp.log(jnp.sum(jnp.exp(e), axis=1, keepdims=True))


_post_tc = pl.pallas_call(
    _post_body,
    grid=(GRID,),
    in_specs=[
        pl.BlockSpec((NW, BN), lambda i: (0, i)),
        pl.BlockSpec((NC, BN, D), lambda i: (0, i, 0)),
        pl.BlockSpec((BN, D), lambda i: (i, 0)),
    ],
    out_specs=pl.BlockSpec((BN, D), lambda i: (i, 0)),
    out_shape=jax.ShapeDtypeStruct((NP, D), jnp.float32),
)


def kernel(x, edge_index, W1, b1, W2, b2):
    row = edge_index[0]
    col = edge_index[1]
    xp = jnp.pad(x, ((0, NP - N), (0, 0)))
    zeros = jnp.zeros((NP, D), jnp.float32)
    hist = _deg_sc(row)
    h1p = _pre_tc(hist, xp, W1, b1.reshape(1, D))
    p1 = _scatter_sc(h1p, row, col, zeros)
    h2p = _mid_tc(hist, p1, h1p, W2, b2.reshape(1, D))
    p2 = _scatter_sc(h2p, row, col, zeros)
    return _post_tc(hist, p2, h2p)[:N]


# speedup vs baseline: 24.3883x; 1.4659x over previous
import functools

import jax
import jax.numpy as jnp
from jax import lax
from jax.experimental import pallas as pl
from jax.experimental.pallas import tpu as pltpu
from jax.experimental.pallas import tpu_sc as plsc

N = 10000
NP = 10240
E = 320000
D = 128

NC = 2
NS = 16
NW = NC * NS

E_TILE = E // NW
CHUNK = 128
N_FULL = E_TILE // CHUNK
REM = E_TILE - N_FULL * CHUNK

ROWS_SUB = NP // NS

_mesh = plsc.VectorSubcoreMesh(core_axis_name="c", subcore_axis_name="s")


@functools.partial(
    pl.kernel,
    out_type=jax.ShapeDtypeStruct((NW, NP), jnp.float32),
    mesh=_mesh,
    compiler_params=pltpu.CompilerParams(needs_layout_passes=False),
    scratch_types=[
        pltpu.VMEM((CHUNK,), jnp.int32),
        pltpu.VMEM((REM,), jnp.int32),
        pltpu.VMEM((NP,), jnp.float32),
    ],
)
def _deg_sc(row_hbm, out_hbm, idx_v, ridx_v, hist_v):
    c = lax.axis_index("c")
    s = lax.axis_index("s")
    wid = s * NC + c
    base = wid * E_TILE

    zeros16 = jnp.zeros((16,), jnp.float32)

    def zbody(i, carry):
        hist_v[pl.ds(i * 16, 16)] = zeros16
        return carry

    lax.fori_loop(0, NP // 16, zbody, 0)

    ones16 = jnp.ones((16,), jnp.float32)

    def cbody(j, carry):
        pltpu.sync_copy(row_hbm.at[pl.ds(base + j * CHUNK, CHUNK)], idx_v)
        for k in range(CHUNK // 16):
            idx16 = idx_v[pl.ds(k * 16, 16)]
            plsc.addupdate_scatter(hist_v, [idx16], ones16)
        return carry

    lax.fori_loop(0, N_FULL, cbody, 0)

    pltpu.sync_copy(row_hbm.at[pl.ds(base + N_FULL * CHUNK, REM)], ridx_v)
    plsc.addupdate_scatter(hist_v, [ridx_v[...]], ones16)

    pltpu.sync_copy(hist_v, out_hbm.at[wid])


N_PAIR = N_FULL // 2 - 1


@functools.partial(
    pl.kernel,
    out_type=jax.ShapeDtypeStruct((NC, NP, D), jnp.float32),
    mesh=_mesh,
    scratch_types=[
        pltpu.VMEM_SHARED((NP, D), jnp.float32),
        pltpu.VMEM((CHUNK,), jnp.int32),
        pltpu.VMEM((CHUNK,), jnp.int32),
        pltpu.VMEM((CHUNK,), jnp.int32),
        pltpu.VMEM((CHUNK,), jnp.int32),
        pltpu.VMEM((CHUNK, D), jnp.float32),
        pltpu.VMEM((CHUNK, D), jnp.float32),
        pltpu.VMEM((REM,), jnp.int32),
        pltpu.VMEM((REM,), jnp.int32),
        pltpu.VMEM((REM, D), jnp.float32),
        pltpu.SemaphoreType.DMA,
        pltpu.SemaphoreType.DMA,
    ],
)
def _scatter_sc(hp_hbm, row_hbm, col_hbm, zeros_hbm, out_hbm,
                acc_sh, row0, col0, row1, col1, msg0, msg1,
                rrow_v, rcol_v, rmsg_v, sem0, sem1):
    c = lax.axis_index("c")
    s = lax.axis_index("s")
    wid = s * NC + c
    base = wid * E_TILE
    r0 = s * ROWS_SUB

    pltpu.sync_copy(zeros_hbm.at[pl.ds(r0, ROWS_SUB)], acc_sh.at[pl.ds(r0, ROWS_SUB)])
    plsc.subcore_barrier()

    def load_idx(j, rv, cv):
        eb = base + j * CHUNK
        pltpu.sync_copy(row_hbm.at[pl.ds(eb, CHUNK)], rv)
        pltpu.sync_copy(col_hbm.at[pl.ds(eb, CHUNK)], cv)

    load_idx(0, row0, col0)
    pltpu.async_copy(hp_hbm.at[row0], msg0, sem0)

    def pair_body(p, carry):
        load_idx(2 * p + 1, row1, col1)
        pltpu.async_copy(hp_hbm.at[row1], msg1, sem1)
        pltpu.make_async_copy(hp_hbm.at[row0], msg0, sem0).wait()
        pltpu.sync_copy(msg0, acc_sh.at[col0], add=True)
        load_idx(2 * p + 2, row0, col0)
        pltpu.async_copy(hp_hbm.at[row0], msg0, sem0)
        pltpu.make_async_copy(hp_hbm.at[row1], msg1, sem1).wait()
        pltpu.sync_copy(msg1, acc_sh.at[col1], add=True)
        return carry

    lax.fori_loop(0, N_PAIR, pair_body, 0)

    load_idx(N_FULL - 1, row1, col1)
    pltpu.async_copy(hp_hbm.at[row1], msg1, sem1)
    pltpu.make_async_copy(hp_hbm.at[row0], msg0, sem0).wait()
    pltpu.sync_copy(msg0, acc_sh.at[col0], add=True)
    eb = base + N_FULL * CHUNK
    pltpu.sync_copy(row_hbm.at[pl.ds(eb, REM)], rrow_v)
    pltpu.sync_copy(col_hbm.at[pl.ds(eb, REM)], rcol_v)
    pltpu.async_copy(hp_hbm.at[rrow_v], rmsg_v, sem0)
    pltpu.make_async_copy(hp_hbm.at[row1], msg1, sem1).wait()
    pltpu.sync_copy(msg1, acc_sh.at[col1], add=True)
    pltpu.make_async_copy(hp_hbm.at[rrow_v], rmsg_v, sem0).wait()
    pltpu.sync_copy(rmsg_v, acc_sh.at[rcol_v], add=True)

    plsc.subcore_barrier()
    pltpu.sync_copy(acc_sh.at[pl.ds(r0, ROWS_SUB)], out_hbm.at[c, pl.ds(r0, ROWS_SUB)])


BN = 1024
GRID = NP // BN


def _dis_from_hist(hist_blk):
    deg = jnp.sum(hist_blk, axis=0) + 1.0
    return lax.rsqrt(deg)


def _pre_body(hist_ref, x_ref, w1_ref, b1_ref, out_ref):
    dis = _dis_from_hist(hist_ref[...])
    h = lax.dot_general(x_ref[...], w1_ref[...], (((1,), (1,)), ((), ())),
                        preferred_element_type=jnp.float32) + b1_ref[...]
    out_ref[...] = dis[:, None] * h


_pre_tc = pl.pallas_call(
    _pre_body,
    grid=(GRID,),
    in_specs=[
        pl.BlockSpec((NW, BN), lambda i: (0, i)),
        pl.BlockSpec((BN, D), lambda i: (i, 0)),
        pl.BlockSpec((D, D), lambda i: (0, 0)),
        pl.BlockSpec((1, D), lambda i: (0, 0)),
    ],
    out_specs=pl.BlockSpec((BN, D), lambda i: (i, 0)),
    out_shape=jax.ShapeDtypeStruct((NP, D), jnp.float32),
)


def _mid_body(hist_ref, p_ref, hp_ref, w2_ref, b2_ref, out_ref):
    dis = _dis_from_hist(hist_ref[...])[:, None]
    sacc = p_ref[0] + p_ref[1] + hp_ref[...]
    y = jnp.maximum(dis * sacc, 0.0)
    h2 = lax.dot_general(y, w2_ref[...], (((1,), (1,)), ((), ())),
                         preferred_element_type=jnp.float32) + b2_ref[...]
    out_ref[...] = dis * h2


_mid_tc = pl.pallas_call(
    _mid_body,
    grid=(GRID,),
    in_specs=[
        pl.BlockSpec((NW, BN), lambda i: (0, i)),
        pl.BlockSpec((NC, BN, D), lambda i: (0, i, 0)),
        pl.BlockSpec((BN, D), lambda i: (i, 0)),
        pl.BlockSpec((D, D), lambda i: (0, 0)),
        pl.BlockSpec((1, D), lambda i: (0, 0)),
    ],
    out_specs=pl.BlockSpec((BN, D), lambda i: (i, 0)),
    out_shape=jax.ShapeDtypeStruct((NP, D), jnp.float32),
)


def _post_body(hist_ref, p_ref, hp_ref, out_ref):
    dis = _dis_from_hist(hist_ref[...])[:, None]
    o = dis * (p_ref[0] + p_ref[1] + hp_ref[...])
    m = jnp.max(o, axis=1, keepdims=True)
    e = o - m
    out_ref[...] = e - jnp.log(jnp.sum(jnp.exp(e), axis=1, keepdims=True))


_post_tc = pl.pallas_call(
    _post_body,
    grid=(GRID,),
    in_specs=[
        pl.BlockSpec((NW, BN), lambda i: (0, i)),
        pl.BlockSpec((NC, BN, D), lambda i: (0, i, 0)),
        pl.BlockSpec((BN, D), lambda i: (i, 0)),
    ],
    out_specs=pl.BlockSpec((BN, D), lambda i: (i, 0)),
    out_shape=jax.ShapeDtypeStruct((NP, D), jnp.float32),
)


def kernel(x, edge_index, W1, b1, W2, b2):
    row = edge_index[0]
    col = edge_index[1]
    xp = jnp.pad(x, ((0, NP - N), (0, 0)))
    zeros = jnp.zeros((NP, D), jnp.float32)
    hist = _deg_sc(row)
    h1p = _pre_tc(hist, xp, W1, b1.reshape(1, D))
    p1 = _scatter_sc(h1p, row, col, zeros)
    h2p = _mid_tc(hist, p1, h1p, W2, b2.reshape(1, D))
    p2 = _scatter_sc(h2p, row, col, zeros)
    return _post_tc(hist, p2, h2p)[:N]


# speedup vs baseline: 28.4533x; 1.1667x over previous
import functools

import jax
import jax.numpy as jnp
from jax import lax
from jax.experimental import pallas as pl
from jax.experimental.pallas import tpu as pltpu
from jax.experimental.pallas import tpu_sc as plsc

N = 10000
NP = 10240
E = 320000
D = 128

NC = 2
NS = 16
NW = NC * NS

E_TILE = E // NW
CHUNK = 128
NCHUNK = 79
E_TILE_P = NCHUNK * CHUNK
N_PAIR = NCHUNK // 2

ROWS_SUB = NP // NS

_mesh = plsc.VectorSubcoreMesh(core_axis_name="c", subcore_axis_name="s")


@functools.partial(
    pl.kernel,
    out_type=jax.ShapeDtypeStruct((NW, NP), jnp.float32),
    mesh=_mesh,
    compiler_params=pltpu.CompilerParams(needs_layout_passes=False),
    scratch_types=[
        pltpu.VMEM((NCHUNK, CHUNK), jnp.int32),
        pltpu.VMEM((NP,), jnp.float32),
    ],
)
def _deg_sc(row_hbm, out_hbm, idx_all, hist_v):
    c = lax.axis_index("c")
    s = lax.axis_index("s")
    wid = s * NC + c

    pltpu.sync_copy(row_hbm.at[wid], idx_all)

    zeros16 = jnp.zeros((16,), jnp.float32)

    def zbody(i, carry):
        hist_v[pl.ds(i * 16, 16)] = zeros16
        return carry

    lax.fori_loop(0, NP // 16, zbody, 0)

    ones16 = jnp.ones((16,), jnp.float32)

    def cbody(j, carry):
        for k in range(CHUNK // 16):
            idx16 = idx_all[j, pl.ds(k * 16, 16)]
            plsc.addupdate_scatter(hist_v, [idx16], ones16)
        return carry

    lax.fori_loop(0, NCHUNK, cbody, 0)

    pltpu.sync_copy(hist_v, out_hbm.at[wid])


@functools.partial(
    pl.kernel,
    out_type=jax.ShapeDtypeStruct((NC, NP, D), jnp.float32),
    mesh=_mesh,
    scratch_types=[
        pltpu.VMEM_SHARED((NP, D), jnp.float32),
        pltpu.VMEM((CHUNK,), jnp.int32),
        pltpu.VMEM((CHUNK,), jnp.int32),
        pltpu.VMEM((CHUNK,), jnp.int32),
        pltpu.VMEM((CHUNK,), jnp.int32),
        pltpu.VMEM((CHUNK, D), jnp.float32),
        pltpu.VMEM((CHUNK, D), jnp.float32),
        pltpu.SemaphoreType.DMA,
        pltpu.SemaphoreType.DMA,
        pltpu.SemaphoreType.DMA,
        pltpu.SemaphoreType.DMA,
        pltpu.SemaphoreType.DMA,
    ],
)
def _scatter_sc(hp_hbm, row_hbm, col_hbm, zeros_hbm, out_hbm,
                acc_sh, row0, col0, row1, col1, msg0, msg1,
                sem0, sem1, isem0, isem1, lsem):
    c = lax.axis_index("c")
    s = lax.axis_index("s")
    wid = s * NC + c
    r0 = s * ROWS_SUB

    cbase = wid * NCHUNK

    def idx_load(j, rv, cv, isem):
        pltpu.async_copy(row_hbm.at[cbase + j], rv, isem)
        pltpu.async_copy(col_hbm.at[cbase + j], cv, isem)

    def idx_wait(j, rv, cv, isem):
        pltpu.make_async_copy(row_hbm.at[cbase + j], rv, isem).wait()
        pltpu.make_async_copy(col_hbm.at[cbase + j], cv, isem).wait()

    zld = pltpu.async_copy(zeros_hbm.at[pl.ds(r0, ROWS_SUB)],
                           acc_sh.at[pl.ds(r0, ROWS_SUB)], lsem)
    idx_load(0, row0, col0, isem0)
    idx_wait(0, row0, col0, isem0)
    pltpu.async_copy(hp_hbm.at[row0], msg0, sem0)
    idx_load(1, row1, col1, isem1)
    zld.wait()
    plsc.subcore_barrier()

    def pair_body(p, carry):
        j0 = 2 * p
        idx_wait(j0 + 1, row1, col1, isem1)
        pltpu.async_copy(hp_hbm.at[row1], msg1, sem1)
        pltpu.make_async_copy(hp_hbm.at[row0], msg0, sem0).wait()
        pltpu.sync_copy(msg0, acc_sh.at[col0], add=True)
        idx_load(j0 + 2, row0, col0, isem0)
        pltpu.make_async_copy(hp_hbm.at[row1], msg1, sem1).wait()
        pltpu.sync_copy(msg1, acc_sh.at[col1], add=True)
        idx_wait(j0 + 2, row0, col0, isem0)
        pltpu.async_copy(hp_hbm.at[row0], msg0, sem0)
        idx_load(j0 + 3, row1, col1, isem1)
        return carry

    lax.fori_loop(0, N_PAIR - 1, pair_body, 0)

    j = NCHUNK - 3
    idx_wait(j + 1, row1, col1, isem1)
    pltpu.async_copy(hp_hbm.at[row1], msg1, sem1)
    pltpu.make_async_copy(hp_hbm.at[row0], msg0, sem0).wait()
    pltpu.sync_copy(msg0, acc_sh.at[col0], add=True)
    idx_load(j + 2, row0, col0, isem0)
    pltpu.make_async_copy(hp_hbm.at[row1], msg1, sem1).wait()
    pltpu.sync_copy(msg1, acc_sh.at[col1], add=True)
    idx_wait(j + 2, row0, col0, isem0)
    pltpu.async_copy(hp_hbm.at[row0], msg0, sem0)
    pltpu.make_async_copy(hp_hbm.at[row0], msg0, sem0).wait()
    pltpu.sync_copy(msg0, acc_sh.at[col0], add=True)

    plsc.subcore_barrier()
    pltpu.sync_copy(acc_sh.at[pl.ds(r0, ROWS_SUB)], out_hbm.at[c, pl.ds(r0, ROWS_SUB)])


BN = 1024
GRID = NP // BN


def _dis_from_hist(hist_blk):
    deg = jnp.sum(hist_blk, axis=0) + 1.0
    return lax.rsqrt(deg)


def _pre_body(hist_ref, x_ref, w1_ref, b1_ref, out_ref):
    dis = _dis_from_hist(hist_ref[...])
    h = lax.dot_general(x_ref[...], w1_ref[...], (((1,), (1,)), ((), ())),
                        preferred_element_type=jnp.float32) + b1_ref[...]
    out_ref[...] = dis[:, None] * h


_pre_tc = pl.pallas_call(
    _pre_body,
    grid=(GRID,),
    in_specs=[
        pl.BlockSpec((NW, BN), lambda i: (0, i)),
        pl.BlockSpec((BN, D), lambda i: (i, 0)),
        pl.BlockSpec((D, D), lambda i: (0, 0)),
        pl.BlockSpec((1, D), lambda i: (0, 0)),
    ],
    out_specs=pl.BlockSpec((BN, D), lambda i: (i, 0)),
    out_shape=jax.ShapeDtypeStruct((NP, D), jnp.float32),
)


def _mid_body(hist_ref, p_ref, hp_ref, w2_ref, b2_ref, out_ref):
    dis = _dis_from_hist(hist_ref[...])[:, None]
    sacc = p_ref[0] + p_ref[1] + hp_ref[...]
    y = jnp.maximum(dis * sacc, 0.0)
    h2 = lax.dot_general(y, w2_ref[...], (((1,), (1,)), ((), ())),
                         preferred_element_type=jnp.float32) + b2_ref[...]
    out_ref[...] = dis * h2


_mid_tc = pl.pallas_call(
    _mid_body,
    grid=(GRID,),
    in_specs=[
        pl.BlockSpec((NW, BN), lambda i: (0, i)),
        pl.BlockSpec((NC, BN, D), lambda i: (0, i, 0)),
        pl.BlockSpec((BN, D), lambda i: (i, 0)),
        pl.BlockSpec((D, D), lambda i: (0, 0)),
        pl.BlockSpec((1, D), lambda i: (0, 0)),
    ],
    out_specs=pl.BlockSpec((BN, D), lambda i: (i, 0)),
    out_shape=jax.ShapeDtypeStruct((NP, D), jnp.float32),
)


def _post_body(hist_ref, p_ref, hp_ref, out_ref):
    dis = _dis_from_hist(hist_ref[...])[:, None]
    o = dis * (p_ref[0] + p_ref[1] + hp_ref[...])
    m = jnp.max(o, axis=1, keepdims=True)
    e = o - m
    out_ref[...] = e - jnp.log(jnp.sum(jnp.exp(e), axis=1, keepdims=True))


_post_tc = pl.pallas_call(
    _post_body,
    grid=(GRID,),
    in_specs=[
        pl.BlockSpec((NW, BN), lambda i: (0, i)),
        pl.BlockSpec((NC, BN, D), lambda i: (0, i, 0)),
        pl.BlockSpec((BN, D), lambda i: (i, 0)),
    ],
    out_specs=pl.BlockSpec((BN, D), lambda i: (i, 0)),
    out_shape=jax.ShapeDtypeStruct((NP, D), jnp.float32),
)


def _pad_edges(v):
    vt = v.reshape(NW, E_TILE)
    padv = N + (jnp.arange(E_TILE_P - E_TILE, dtype=jnp.int32) % (NP - N))
    padv = jnp.broadcast_to(padv, (NW, E_TILE_P - E_TILE))
    return jnp.concatenate([vt, padv], axis=1).reshape(NW, NCHUNK, CHUNK)


def kernel(x, edge_index, W1, b1, W2, b2):
    row = _pad_edges(edge_index[0])
    col = _pad_edges(edge_index[1])
    row2 = row.reshape(NW * NCHUNK, CHUNK)
    col2 = col.reshape(NW * NCHUNK, CHUNK)
    xp = jnp.pad(x, ((0, NP - N), (0, 0)))
    zeros = jnp.zeros((NP, D), jnp.float32)
    hist = _deg_sc(row)
    h1p = _pre_tc(hist, xp, W1, b1.reshape(1, D))
    p1 = _scatter_sc(h1p, row2, col2, zeros)
    h2p = _mid_tc(hist, p1, h1p, W2, b2.reshape(1, D))
    p2 = _scatter_sc(h2p, row2, col2, zeros)
    return _post_tc(hist, p2, h2p)[:N]


# speedup vs baseline: 30.8571x; 1.0845x over previous
import functools

import jax
import jax.numpy as jnp
from jax import lax
from jax.experimental import pallas as pl
from jax.experimental.pallas import tpu as pltpu
from jax.experimental.pallas import tpu_sc as plsc

N = 10000
NP = 10240
E = 320000
D = 128

NC = 2
NS = 16
NW = NC * NS

E_TILE = E // NW
CHUNK = 128
NCHUNK = 79
E_TILE_P = NCHUNK * CHUNK
N_PAIR = NCHUNK // 2

ROWS_SUB = NP // NS

_mesh = plsc.VectorSubcoreMesh(core_axis_name="c", subcore_axis_name="s")


@functools.partial(
    pl.kernel,
    out_type=jax.ShapeDtypeStruct((NW, NP), jnp.float32),
    mesh=_mesh,
    compiler_params=pltpu.CompilerParams(needs_layout_passes=False),
    scratch_types=[
        pltpu.VMEM((NCHUNK, CHUNK), jnp.int32),
        pltpu.VMEM((NP,), jnp.float32),
    ],
)
def _deg_sc(row_hbm, out_hbm, idx_all, hist_v):
    c = lax.axis_index("c")
    s = lax.axis_index("s")
    wid = s * NC + c

    pltpu.sync_copy(row_hbm.at[wid], idx_all)

    zeros16 = jnp.zeros((16,), jnp.float32)

    def zbody(i, carry):
        hist_v[pl.ds(i * 16, 16)] = zeros16
        return carry

    lax.fori_loop(0, NP // 16, zbody, 0)

    ones16 = jnp.ones((16,), jnp.float32)

    def cbody(j, carry):
        for k in range(CHUNK // 16):
            idx16 = idx_all[j, pl.ds(k * 16, 16)]
            plsc.addupdate_scatter(hist_v, [idx16], ones16)
        return carry

    lax.fori_loop(0, NCHUNK, cbody, 0)

    pltpu.sync_copy(hist_v, out_hbm.at[wid])


@functools.partial(
    pl.kernel,
    out_type=jax.ShapeDtypeStruct((NC, NP, D), jnp.float32),
    mesh=_mesh,
    scratch_types=[
        pltpu.VMEM_SHARED((NP, D), jnp.float32),
        pltpu.VMEM((2, CHUNK), jnp.int32),
        pltpu.VMEM((2, CHUNK), jnp.int32),
        pltpu.VMEM((CHUNK, D), jnp.float32),
        pltpu.VMEM((CHUNK, D), jnp.float32),
        pltpu.SemaphoreType.DMA,
        pltpu.SemaphoreType.DMA,
        pltpu.SemaphoreType.DMA,
        pltpu.SemaphoreType.DMA,
        pltpu.SemaphoreType.DMA,
    ],
)
def _scatter_sc(hp_hbm, rc_hbm, zeros_hbm, out_hbm,
                acc_sh, rc0, rc1, msg0, msg1,
                sem0, sem1, isem0, isem1, lsem):
    c = lax.axis_index("c")
    s = lax.axis_index("s")
    wid = s * NC + c
    r0 = s * ROWS_SUB
    cbase = wid * NCHUNK

    def idx_load(j, rc, isem):
        pltpu.async_copy(rc_hbm.at[cbase + j], rc, isem)

    def idx_wait(j, rc, isem):
        pltpu.make_async_copy(rc_hbm.at[cbase + j], rc, isem).wait()

    def gather(rc, msg, sem):
        pltpu.async_copy(hp_hbm.at[rc.at[0]], msg, sem)

    def gather_wait(rc, msg, sem):
        pltpu.make_async_copy(hp_hbm.at[rc.at[0]], msg, sem).wait()

    def scatter(rc, msg):
        pltpu.sync_copy(msg, acc_sh.at[rc.at[1]], add=True)

    zld = pltpu.async_copy(zeros_hbm.at[pl.ds(r0, ROWS_SUB)],
                           acc_sh.at[pl.ds(r0, ROWS_SUB)], lsem)
    idx_load(0, rc0, isem0)
    idx_wait(0, rc0, isem0)
    gather(rc0, msg0, sem0)
    idx_load(1, rc1, isem1)
    zld.wait()
    plsc.subcore_barrier()

    def pair_body(p, carry):
        j0 = 2 * p
        idx_wait(j0 + 1, rc1, isem1)
        gather(rc1, msg1, sem1)
        gather_wait(rc0, msg0, sem0)
        scatter(rc0, msg0)
        idx_load(j0 + 2, rc0, isem0)
        idx_wait(j0 + 2, rc0, isem0)
        gather(rc0, msg0, sem0)
        gather_wait(rc1, msg1, sem1)
        scatter(rc1, msg1)
        idx_load(j0 + 3, rc1, isem1)
        return carry

    lax.fori_loop(0, N_PAIR - 1, pair_body, 0)

    idx_wait(NCHUNK - 2, rc1, isem1)
    gather(rc1, msg1, sem1)
    gather_wait(rc0, msg0, sem0)
    scatter(rc0, msg0)
    idx_load(NCHUNK - 1, rc0, isem0)
    idx_wait(NCHUNK - 1, rc0, isem0)
    gather(rc0, msg0, sem0)
    gather_wait(rc1, msg1, sem1)
    scatter(rc1, msg1)
    gather_wait(rc0, msg0, sem0)
    scatter(rc0, msg0)

    plsc.subcore_barrier()
    pltpu.sync_copy(acc_sh.at[pl.ds(r0, ROWS_SUB)], out_hbm.at[c, pl.ds(r0, ROWS_SUB)])


BN = 1024
GRID = NP // BN


def _dis_from_hist(hist_blk):
    deg = jnp.sum(hist_blk, axis=0) + 1.0
    return lax.rsqrt(deg)


def _pre_body(hist_ref, x_ref, w1_ref, b1_ref, out_ref):
    dis = _dis_from_hist(hist_ref[...])
    h = lax.dot_general(x_ref[...], w1_ref[...], (((1,), (1,)), ((), ())),
                        preferred_element_type=jnp.float32) + b1_ref[...]
    out_ref[...] = dis[:, None] * h


_pre_tc = pl.pallas_call(
    _pre_body,
    grid=(GRID,),
    in_specs=[
        pl.BlockSpec((NW, BN), lambda i: (0, i)),
        pl.BlockSpec((BN, D), lambda i: (i, 0)),
        pl.BlockSpec((D, D), lambda i: (0, 0)),
        pl.BlockSpec((1, D), lambda i: (0, 0)),
    ],
    out_specs=pl.BlockSpec((BN, D), lambda i: (i, 0)),
    out_shape=jax.ShapeDtypeStruct((NP, D), jnp.float32),
)


def _mid_body(hist_ref, p_ref, hp_ref, w2_ref, b2_ref, out_ref):
    dis = _dis_from_hist(hist_ref[...])[:, None]
    sacc = p_ref[0] + p_ref[1] + hp_ref[...]
    y = jnp.maximum(dis * sacc, 0.0)
    h2 = lax.dot_general(y, w2_ref[...], (((1,), (1,)), ((), ())),
                         preferred_element_type=jnp.float32) + b2_ref[...]
    out_ref[...] = dis * h2


_mid_tc = pl.pallas_call(
    _mid_body,
    grid=(GRID,),
    in_specs=[
        pl.BlockSpec((NW, BN), lambda i: (0, i)),
        pl.BlockSpec((NC, BN, D), lambda i: (0, i, 0)),
        pl.BlockSpec((BN, D), lambda i: (i, 0)),
        pl.BlockSpec((D, D), lambda i: (0, 0)),
        pl.BlockSpec((1, D), lambda i: (0, 0)),
    ],
    out_specs=pl.BlockSpec((BN, D), lambda i: (i, 0)),
    out_shape=jax.ShapeDtypeStruct((NP, D), jnp.float32),
)


def _post_body(hist_ref, p_ref, hp_ref, out_ref):
    dis = _dis_from_hist(hist_ref[...])[:, None]
    o = dis * (p_ref[0] + p_ref[1] + hp_ref[...])
    m = jnp.max(o, axis=1, keepdims=True)
    e = o - m
    out_ref[...] = e - jnp.log(jnp.sum(jnp.exp(e), axis=1, keepdims=True))


_post_tc = pl.pallas_call(
    _post_body,
    grid=(GRID,),
    in_specs=[
        pl.BlockSpec((NW, BN), lambda i: (0, i)),
        pl.BlockSpec((NC, BN, D), lambda i: (0, i, 0)),
        pl.BlockSpec((BN, D), lambda i: (i, 0)),
    ],
    out_specs=pl.BlockSpec((BN, D), lambda i: (i, 0)),
    out_shape=jax.ShapeDtypeStruct((NP, D), jnp.float32),
)


def _pad_edges(v):
    vt = v.reshape(NW, E_TILE)
    padv = N + (jnp.arange(E_TILE_P - E_TILE, dtype=jnp.int32) % (NP - N))
    padv = jnp.broadcast_to(padv, (NW, E_TILE_P - E_TILE))
    return jnp.concatenate([vt, padv], axis=1).reshape(NW, NCHUNK, CHUNK)


def kernel(x, edge_index, W1, b1, W2, b2):
    row = _pad_edges(edge_index[0])
    col = _pad_edges(edge_index[1])
    rc = jnp.stack([row, col], axis=2).reshape(NW * NCHUNK, 2, CHUNK)
    xp = jnp.pad(x, ((0, NP - N), (0, 0)))
    zeros = jnp.zeros((NP, D), jnp.float32)
    hist = _deg_sc(row)
    h1p = _pre_tc(hist, xp, W1, b1.reshape(1, D))
    p1 = _scatter_sc(h1p, rc, zeros)
    h2p = _mid_tc(hist, p1, h1p, W2, b2.reshape(1, D))
    p2 = _scatter_sc(h2p, rc, zeros)
    return _post_tc(hist, p2, h2p)[:N]


# speedup vs baseline: 34.4470x; 1.1163x over previous
import functools

import jax
import jax.numpy as jnp
from jax import lax
from jax.experimental import pallas as pl
from jax.experimental.pallas import tpu as pltpu
from jax.experimental.pallas import tpu_sc as plsc

N = 10000
NP = 10240
E = 320000
D = 128

NC = 2
NS = 16
NW = NC * NS

E_TILE = E // NW
CHUNK = 128
NCHUNK = 79
E_TILE_P = NCHUNK * CHUNK
N_PAIR = NCHUNK // 2

ROWS_SUB = NP // NS

_mesh = plsc.VectorSubcoreMesh(core_axis_name="c", subcore_axis_name="s")


@functools.partial(
    pl.kernel,
    out_type=jax.ShapeDtypeStruct((NW, NP), jnp.float32),
    mesh=_mesh,
    compiler_params=pltpu.CompilerParams(needs_layout_passes=False),
    scratch_types=[
        pltpu.VMEM((NCHUNK, CHUNK), jnp.int32),
        pltpu.VMEM((NP,), jnp.float32),
    ],
)
def _deg_sc(row_hbm, out_hbm, idx_all, hist_v):
    c = lax.axis_index("c")
    s = lax.axis_index("s")
    wid = s * NC + c

    pltpu.sync_copy(row_hbm.at[wid], idx_all)

    zeros16 = jnp.zeros((16,), jnp.float32)

    def zbody(i, carry):
        hist_v[pl.ds(i * 16, 16)] = zeros16
        return carry

    lax.fori_loop(0, NP // 16, zbody, 0)

    ones16 = jnp.ones((16,), jnp.float32)

    def cbody(j, carry):
        for k in range(CHUNK // 16):
            idx16 = idx_all[j, pl.ds(k * 16, 16)]
            plsc.addupdate_scatter(hist_v, [idx16], ones16)
        return carry

    lax.fori_loop(0, NCHUNK, cbody, 0)

    pltpu.sync_copy(hist_v, out_hbm.at[wid])


N_QUAD = 19


@functools.partial(
    pl.kernel,
    out_type=jax.ShapeDtypeStruct((NC, NP, D), jnp.float32),
    mesh=_mesh,
    scratch_types=[
        pltpu.VMEM_SHARED((NP, D), jnp.float32),
        pltpu.VMEM((2, CHUNK), jnp.int32),
        pltpu.VMEM((2, CHUNK), jnp.int32),
        pltpu.VMEM((2, CHUNK), jnp.int32),
        pltpu.VMEM((2, CHUNK), jnp.int32),
        pltpu.VMEM((CHUNK, D), jnp.float32),
        pltpu.VMEM((CHUNK, D), jnp.float32),
        pltpu.SemaphoreType.DMA,
        pltpu.SemaphoreType.DMA,
        pltpu.SemaphoreType.DMA,
        pltpu.SemaphoreType.DMA,
        pltpu.SemaphoreType.DMA,
        pltpu.SemaphoreType.DMA,
        pltpu.SemaphoreType.DMA,
    ],
)
def _scatter_sc(hp_hbm, rc_hbm, zeros_hbm, out_hbm,
                acc_sh, rc0, rc1, rc2, rc3, msg0, msg1,
                sem0, sem1, isem0, isem1, isem2, isem3, lsem):
    c = lax.axis_index("c")
    s = lax.axis_index("s")
    wid = s * NC + c
    r0 = s * ROWS_SUB
    cbase = wid * NCHUNK

    def idx_load(j, rc, isem):
        pltpu.async_copy(rc_hbm.at[cbase + j], rc, isem)

    def idx_wait(j, rc, isem):
        pltpu.make_async_copy(rc_hbm.at[cbase + j], rc, isem).wait()

    def gather(rc, msg, sem):
        pltpu.async_copy(hp_hbm.at[rc.at[0]], msg, sem)

    def gather_wait(rc, msg, sem):
        pltpu.make_async_copy(hp_hbm.at[rc.at[0]], msg, sem).wait()

    def scatter(rc, msg):
        pltpu.sync_copy(msg, acc_sh.at[rc.at[1]], add=True)

    zld = pltpu.async_copy(zeros_hbm.at[pl.ds(r0, ROWS_SUB)],
                           acc_sh.at[pl.ds(r0, ROWS_SUB)], lsem)
    idx_load(0, rc0, isem0)
    idx_wait(0, rc0, isem0)
    gather(rc0, msg0, sem0)
    idx_load(1, rc1, isem1)
    idx_load(2, rc2, isem2)
    zld.wait()
    plsc.subcore_barrier()

    def quad_body(q, carry):
        j0 = 4 * q
        idx_wait(j0 + 1, rc1, isem1)
        gather(rc1, msg1, sem1)
        gather_wait(rc0, msg0, sem0)
        scatter(rc0, msg0)
        idx_load(j0 + 3, rc3, isem3)
        idx_wait(j0 + 2, rc2, isem2)
        gather(rc2, msg0, sem0)
        gather_wait(rc1, msg1, sem1)
        scatter(rc1, msg1)
        idx_load(j0 + 4, rc0, isem0)
        idx_wait(j0 + 3, rc3, isem3)
        gather(rc3, msg1, sem1)
        gather_wait(rc2, msg0, sem0)
        scatter(rc2, msg0)
        idx_load(j0 + 5, rc1, isem1)
        idx_wait(j0 + 4, rc0, isem0)
        gather(rc0, msg0, sem0)
        gather_wait(rc3, msg1, sem1)
        scatter(rc3, msg1)
        idx_load(j0 + 6, rc2, isem2)
        return carry

    lax.fori_loop(0, N_QUAD, quad_body, 0)

    idx_wait(NCHUNK - 2, rc1, isem1)
    gather(rc1, msg1, sem1)
    gather_wait(rc0, msg0, sem0)
    scatter(rc0, msg0)
    idx_wait(NCHUNK - 1, rc2, isem2)
    gather(rc2, msg0, sem0)
    gather_wait(rc1, msg1, sem1)
    scatter(rc1, msg1)
    gather_wait(rc2, msg0, sem0)
    scatter(rc2, msg0)

    plsc.subcore_barrier()
    pltpu.sync_copy(acc_sh.at[pl.ds(r0, ROWS_SUB)], out_hbm.at[c, pl.ds(r0, ROWS_SUB)])


BN = 1024
GRID = NP // BN


def _dis_from_hist(hist_blk):
    deg = jnp.sum(hist_blk, axis=0) + 1.0
    return lax.rsqrt(deg)


def _pre_body(hist_ref, x_ref, w1_ref, b1_ref, out_ref):
    dis = _dis_from_hist(hist_ref[...])
    h = lax.dot_general(x_ref[...], w1_ref[...], (((1,), (1,)), ((), ())),
                        preferred_element_type=jnp.float32) + b1_ref[...]
    out_ref[...] = dis[:, None] * h


_pre_tc = pl.pallas_call(
    _pre_body,
    grid=(GRID,),
    in_specs=[
        pl.BlockSpec((NW, BN), lambda i: (0, i)),
        pl.BlockSpec((BN, D), lambda i: (i, 0)),
        pl.BlockSpec((D, D), lambda i: (0, 0)),
        pl.BlockSpec((1, D), lambda i: (0, 0)),
    ],
    out_specs=pl.BlockSpec((BN, D), lambda i: (i, 0)),
    out_shape=jax.ShapeDtypeStruct((NP, D), jnp.float32),
)


def _mid_body(hist_ref, p_ref, hp_ref, w2_ref, b2_ref, out_ref):
    dis = _dis_from_hist(hist_ref[...])[:, None]
    sacc = p_ref[0] + p_ref[1] + hp_ref[...]
    y = jnp.maximum(dis * sacc, 0.0)
    h2 = lax.dot_general(y, w2_ref[...], (((1,), (1,)), ((), ())),
                         preferred_element_type=jnp.float32) + b2_ref[...]
    out_ref[...] = dis * h2


_mid_tc = pl.pallas_call(
    _mid_body,
    grid=(GRID,),
    in_specs=[
        pl.BlockSpec((NW, BN), lambda i: (0, i)),
        pl.BlockSpec((NC, BN, D), lambda i: (0, i, 0)),
        pl.BlockSpec((BN, D), lambda i: (i, 0)),
        pl.BlockSpec((D, D), lambda i: (0, 0)),
        pl.BlockSpec((1, D), lambda i: (0, 0)),
    ],
    out_specs=pl.BlockSpec((BN, D), lambda i: (i, 0)),
    out_shape=jax.ShapeDtypeStruct((NP, D), jnp.float32),
)


def _post_body(hist_ref, p_ref, hp_ref, out_ref):
    dis = _dis_from_hist(hist_ref[...])[:, None]
    o = dis * (p_ref[0] + p_ref[1] + hp_ref[...])
    m = jnp.max(o, axis=1, keepdims=True)
    e = o - m
    out_ref[...] = e - jnp.log(jnp.sum(jnp.exp(e), axis=1, keepdims=True))


_post_tc = pl.pallas_call(
    _post_body,
    grid=(GRID,),
    in_specs=[
        pl.BlockSpec((NW, BN), lambda i: (0, i)),
        pl.BlockSpec((NC, BN, D), lambda i: (0, i, 0)),
        pl.BlockSpec((BN, D), lambda i: (i, 0)),
    ],
    out_specs=pl.BlockSpec((BN, D), lambda i: (i, 0)),
    out_shape=jax.ShapeDtypeStruct((NP, D), jnp.float32),
)


def _pad_edges(v):
    vt = v.reshape(NW, E_TILE)
    padv = N + (jnp.arange(E_TILE_P - E_TILE, dtype=jnp.int32) % (NP - N))
    padv = jnp.broadcast_to(padv, (NW, E_TILE_P - E_TILE))
    return jnp.concatenate([vt, padv], axis=1).reshape(NW, NCHUNK, CHUNK)


def kernel(x, edge_index, W1, b1, W2, b2):
    row = _pad_edges(edge_index[0])
    col = _pad_edges(edge_index[1])
    rc = jnp.stack([row, col], axis=2).reshape(NW * NCHUNK, 2, CHUNK)
    xp = jnp.pad(x, ((0, NP - N), (0, 0)))
    zeros = jnp.zeros((NP, D), jnp.float32)
    hist = _deg_sc(row)
    h1p = _pre_tc(hist, xp, W1, b1.reshape(1, D))
    p1 = _scatter_sc(h1p, rc, zeros)
    h2p = _mid_tc(hist, p1, h1p, W2, b2.reshape(1, D))
    p2 = _scatter_sc(h2p, rc, zeros)
    return _post_tc(hist, p2, h2p)[:N]
